# Initial kernel scaffold; baseline (speedup 1.0000x reference)
import jax
import jax.numpy as jnp
from jax.experimental import pallas as pl


def kernel(x, edge_index, adj_values, W, b):
    raise NotImplementedError("write your pallas kernel here")


# speedup vs baseline: 4.3614x; 4.3614x over previous
import functools

import jax
import jax.numpy as jnp
from jax import lax
from jax.experimental import pallas as pl
from jax.experimental.pallas import tpu as pltpu
from jax.experimental.pallas import tpu_sc as plsc

SMOOTH = 0.5

NC = 2
NS = 16
NW = NC * NS
LANES = 16


def _sc_scatter(x, src, dst, adj, *, n, d, e):
    per_w = e // NW
    K = 80
    n_chunks = per_w // K
    ZC = 16
    stripe_lo = (n // NS) // ZC * ZC
    stripe_hi = n - (NS - 1) * stripe_lo
    d_vecs = d // LANES

    mesh = plsc.VectorSubcoreMesh(core_axis_name="c", subcore_axis_name="s")

    @functools.partial(
        pl.kernel,
        out_type=jax.ShapeDtypeStruct((NC, n, d), jnp.float32),
        mesh=mesh,
        scratch_types=[
            pltpu.VMEM((K,), jnp.int32),
            pltpu.VMEM((K,), jnp.int32),
            pltpu.VMEM((K,), jnp.float32),
            pltpu.VMEM((K, d), jnp.float32),
            pltpu.VMEM((ZC, d), jnp.float32),
            pltpu.VMEM_SHARED((n, d), jnp.float32),
            pltpu.SemaphoreType.DMA,
        ],
    )
    def k(x_hbm, src_hbm, dst_hbm, adj_hbm, out_hbm,
          src_v, dst_v, adj_v, rows_v, zero_v, acc, sem):
        cid = lax.axis_index("c")
        sid = lax.axis_index("s")
        wid = sid * NC + cid

        zvec = jnp.zeros((LANES,), jnp.float32)

        @pl.loop(0, ZC)
        def _zero_rows(r):
            for j in range(d_vecs):
                zero_v[r, pl.ds(j * LANES, LANES)] = zvec

        row_base = pl.multiple_of(sid * stripe_lo, ZC)
        n_zc = jnp.where(sid == NS - 1, stripe_hi // ZC, stripe_lo // ZC)

        @pl.loop(0, n_zc)
        def _zero_acc(t):
            off = pl.multiple_of(row_base + t * ZC, ZC)
            pltpu.sync_copy(zero_v, acc.at[pl.ds(off, ZC)])

        plsc.subcore_barrier()

        @pl.loop(0, n_chunks)
        def _chunk(c):
            base = wid * per_w + c * K
            pltpu.sync_copy(src_hbm.at[pl.ds(base, K)], src_v)
            pltpu.sync_copy(dst_hbm.at[pl.ds(base, K)], dst_v)
            pltpu.sync_copy(adj_hbm.at[pl.ds(base, K)], adj_v)
            pltpu.async_copy(x_hbm.at[src_v], rows_v, sem).wait()

            @pl.loop(0, K // LANES)
            def _scale(g):
                a16 = adj_v[pl.ds(g * LANES, LANES)]
                for t in range(LANES):
                    av = jnp.full((LANES,), a16[t], jnp.float32)
                    row = g * LANES + t
                    for j in range(d_vecs):
                        sl = pl.ds(j * LANES, LANES)
                        rows_v[row, sl] = rows_v[row, sl] * av

            pltpu.sync_copy(rows_v, acc.at[dst_v], add=True)

        plsc.subcore_barrier()

        @pl.loop(0, n_zc)
        def _copy_out(t):
            off = pl.multiple_of(row_base + t * ZC, ZC)
            pltpu.sync_copy(acc.at[pl.ds(off, ZC)],
                            out_hbm.at[cid, pl.ds(off, ZC)])

    return k(x, src, dst, adj)


def _tc_combine_body(x_ref, agg_ref, w_ref, b_ref, out_ref):
    y = x_ref[...] + SMOOTH * (agg_ref[0] + agg_ref[1])
    out = jnp.dot(y, w_ref[...], preferred_element_type=jnp.float32)
    out_ref[...] = out / (1.0 + SMOOTH) + b_ref[...]


def _tc_combine(x, aggp, W, b, *, n, d_in, d_out):
    blk = 1000
    grid = n // blk
    return pl.pallas_call(
        _tc_combine_body,
        grid=(grid,),
        in_specs=[
            pl.BlockSpec((blk, d_in), lambda i: (i, 0)),
            pl.BlockSpec((NC, blk, d_in), lambda i: (0, i, 0)),
            pl.BlockSpec((d_in, d_out), lambda i: (0, 0)),
            pl.BlockSpec((1, d_out), lambda i: (0, 0)),
        ],
        out_specs=pl.BlockSpec((blk, d_out), lambda i: (i, 0)),
        out_shape=jax.ShapeDtypeStruct((n, d_out), jnp.float32),
    )(x, aggp, W, b)


def kernel(x, edge_index, adj_values, W, b):
    n, d_in = x.shape
    d_out = W.shape[1]
    e = edge_index.shape[1]
    src = edge_index[0].astype(jnp.int32)
    dst = edge_index[1].astype(jnp.int32)
    aggp = _sc_scatter(x, src, dst, adj_values, n=n, d=d_in, e=e)
    return _tc_combine(x, aggp, W, b.reshape(1, d_out), n=n, d_in=d_in, d_out=d_out)


# speedup vs baseline: 10.4658x; 2.3997x over previous
import functools

import jax
import jax.numpy as jnp
from jax import lax
from jax.experimental import pallas as pl
from jax.experimental.pallas import tpu as pltpu
from jax.experimental.pallas import tpu_sc as plsc

SMOOTH = 0.5

NC = 2
NS = 16
NW = NC * NS
LANES = 16


def _sc_scatter(x, src, dst, adj, *, n, d, e):
    per_w = e // NW
    K = 80
    n_chunks = per_w // K
    NBUF = 4
    n_groups = n_chunks // NBUF
    n_tail = n_chunks - n_groups * NBUF
    ZC = 16
    stripe_lo = (n // NS) // ZC * ZC
    CP = 208
    d_vecs = d // LANES

    mesh = plsc.VectorSubcoreMesh(core_axis_name="c", subcore_axis_name="s")

    @functools.partial(
        pl.kernel,
        out_type=jax.ShapeDtypeStruct((NC, n, d), jnp.float32),
        mesh=mesh,
        scratch_types=[
            pltpu.VMEM((NBUF * K,), jnp.int32),
            pltpu.VMEM((NBUF * K,), jnp.float32),
            [pltpu.VMEM((K,), jnp.int32) for _ in range(NBUF)],
            [pltpu.VMEM((K, d), jnp.float32) for _ in range(NBUF)],
            pltpu.VMEM((ZC, d), jnp.float32),
            pltpu.VMEM_SHARED((n, d), jnp.float32),
            pltpu.SemaphoreType.DMA,
            [pltpu.SemaphoreType.DMA for _ in range(NBUF)],
            [pltpu.SemaphoreType.DMA for _ in range(NBUF)],
            [pltpu.SemaphoreType.DMA for _ in range(NBUF)],
        ],
    )
    def k(x_hbm, src_hbm, dst_hbm, adj_hbm, out_hbm,
          srcg, adjg, dstb, rows, zero_v, acc, isem, dsems, gsems, ssems):
        cid = lax.axis_index("c")
        sid = lax.axis_index("s")
        wid = sid * NC + cid
        ebase = pl.multiple_of(wid * per_w, 8)

        zvec = jnp.zeros((LANES,), jnp.float32)

        @pl.loop(0, ZC)
        def _zero_rows(r):
            for j in range(d_vecs):
                zero_v[r, pl.ds(j * LANES, LANES)] = zvec

        row_base = pl.multiple_of(sid * stripe_lo, ZC)
        n_zc = jnp.where(sid == NS - 1, (n - (NS - 1) * stripe_lo) // ZC,
                         stripe_lo // ZC)

        @pl.loop(0, n_zc)
        def _zero_acc(t):
            off = pl.multiple_of(row_base + t * ZC, ZC)
            pltpu.sync_copy(zero_v, acc.at[pl.ds(off, ZC)])

        plsc.subcore_barrier()

        def scale_rows(buf, adj_off):
            @pl.loop(0, K // LANES)
            def _scale(q):
                a16 = adjg[pl.ds(adj_off + q * LANES, LANES)]
                for t in range(LANES):
                    av = jnp.full((LANES,), a16[t], jnp.float32)
                    row = q * LANES + t
                    for j in range(d_vecs):
                        sl = pl.ds(j * LANES, LANES)
                        buf[row, sl] = buf[row, sl] * av

        @pl.loop(0, n_groups)
        def _group(g):
            go = pl.multiple_of(ebase + g * (NBUF * K), 8)
            sd = pltpu.async_copy(src_hbm.at[pl.ds(go, NBUF * K)], srcg, isem)
            ad = pltpu.async_copy(adj_hbm.at[pl.ds(go, NBUF * K)], adjg, isem)
            dds, gds = [], []
            for b in range(NBUF):
                @pl.when(g > 0)
                def _drain():
                    pltpu.make_async_copy(rows[b], acc.at[dstb[b]],
                                          ssems[b]).wait()

                eo = pl.multiple_of(ebase + g * (NBUF * K) + b * K, 8)
                dds.append(pltpu.async_copy(dst_hbm.at[pl.ds(eo, K)],
                                            dstb[b], dsems[b]))
                if b == 0:
                    sd.wait()
                gds.append(pltpu.async_copy(
                    x_hbm.at[srcg.at[pl.ds(b * K, K)]], rows[b], gsems[b]))
            ad.wait()
            for b in range(NBUF):
                gds[b].wait()
                scale_rows(rows[b], b * K)
                dds[b].wait()
                pltpu.async_copy(rows[b], acc.at[dstb[b]], ssems[b], add=True)

        for b in range(NBUF):
            pltpu.make_async_copy(rows[b], acc.at[dstb[b]], ssems[b]).wait()

        for t in range(n_tail):
            eo = pl.multiple_of(ebase + (n_groups * NBUF + t) * K, 8)
            pltpu.sync_copy(src_hbm.at[pl.ds(eo, K)], srcg.at[pl.ds(0, K)])
            pltpu.sync_copy(adj_hbm.at[pl.ds(eo, K)], adjg.at[pl.ds(0, K)])
            pltpu.sync_copy(dst_hbm.at[pl.ds(eo, K)], dstb[0])
            pltpu.async_copy(x_hbm.at[srcg.at[pl.ds(0, K)]],
                             rows[0], gsems[0]).wait()
            scale_rows(rows[0], 0)
            pltpu.sync_copy(rows[0], acc.at[dstb[0]], add=True)

        plsc.subcore_barrier()

        for t in range(stripe_lo // CP):
            off = pl.multiple_of(row_base + t * CP, ZC)
            pltpu.sync_copy(acc.at[pl.ds(off, CP)],
                            out_hbm.at[cid, pl.ds(off, CP)])

        @pl.when(sid == NS - 1)
        def _tail():
            off = (NS - 1) * stripe_lo + (stripe_lo // CP) * CP
            pltpu.sync_copy(acc.at[pl.ds(off, n - off)],
                            out_hbm.at[cid, pl.ds(off, n - off)])

    return k(x, src, dst, adj)


def _tc_combine_body(x_ref, agg_ref, w_ref, b_ref, out_ref):
    y = x_ref[...] + SMOOTH * (agg_ref[0] + agg_ref[1])
    out = jnp.dot(y, w_ref[...], preferred_element_type=jnp.float32)
    out_ref[...] = out / (1.0 + SMOOTH) + b_ref[...]


def _tc_combine(x, aggp, W, b, *, n, d_in, d_out):
    blk = 1000
    grid = n // blk
    return pl.pallas_call(
        _tc_combine_body,
        grid=(grid,),
        in_specs=[
            pl.BlockSpec((blk, d_in), lambda i: (i, 0)),
            pl.BlockSpec((NC, blk, d_in), lambda i: (0, i, 0)),
            pl.BlockSpec((d_in, d_out), lambda i: (0, 0)),
            pl.BlockSpec((1, d_out), lambda i: (0, 0)),
        ],
        out_specs=pl.BlockSpec((blk, d_out), lambda i: (i, 0)),
        out_shape=jax.ShapeDtypeStruct((n, d_out), jnp.float32),
    )(x, aggp, W, b)


def kernel(x, edge_index, adj_values, W, b):
    n, d_in = x.shape
    d_out = W.shape[1]
    e = edge_index.shape[1]
    src = edge_index[0].astype(jnp.int32)
    dst = edge_index[1].astype(jnp.int32)
    aggp = _sc_scatter(x, src, dst, adj_values, n=n, d=d_in, e=e)
    return _tc_combine(x, aggp, W, b.reshape(1, d_out), n=n, d_in=d_in, d_out=d_out)


# speedup vs baseline: 12.6690x; 1.2105x over previous
import functools

import jax
import jax.numpy as jnp
from jax import lax
from jax.experimental import pallas as pl
from jax.experimental.pallas import tpu as pltpu
from jax.experimental.pallas import tpu_sc as plsc

SMOOTH = 0.5

NC = 2
NS = 16
NW = NC * NS
LANES = 16


def _sc_scatter(x, src, dst, adj, *, n, d, e):
    per_w = e // NW
    K = 104
    NPH = per_w // K // 6 * 6
    TE = per_w - NPH * K
    ZC = 16
    stripe_lo = (n // NS) // ZC * ZC
    CP = 208
    d_vecs = d // LANES

    mesh = plsc.VectorSubcoreMesh(core_axis_name="c", subcore_axis_name="s")

    @functools.partial(
        pl.kernel,
        out_type=jax.ShapeDtypeStruct((NC, n, d), jnp.float32),
        mesh=mesh,
        scratch_types=[
            [pltpu.VMEM((K,), jnp.int32) for _ in range(3)],
            [pltpu.VMEM((K,), jnp.float32) for _ in range(3)],
            [pltpu.VMEM((1, K), jnp.int32) for _ in range(6)],
            [pltpu.VMEM((K, d), jnp.float32) for _ in range(3)],
            pltpu.VMEM((1, ZC), jnp.int32),
            pltpu.VMEM_SHARED((n, d), jnp.float32),
            [pltpu.SemaphoreType.DMA for _ in range(3)],
            [pltpu.SemaphoreType.DMA for _ in range(6)],
            [pltpu.SemaphoreType.DMA for _ in range(3)],
            [pltpu.SemaphoreType.DMA for _ in range(3)],
        ],
    )
    def k(x_hbm, src_hbm, dst_hbm, adj_hbm, out_hbm,
          srcb, adjb, dstb, rows, dstt, acc, stsem, dsem, gsem, scsem):
        cid = lax.axis_index("c")
        sid = lax.axis_index("s")
        wid = sid * NC + cid
        ebase = pl.multiple_of(wid * per_w, 8)

        zvec = jnp.zeros((LANES,), jnp.float32)

        @pl.loop(0, ZC)
        def _zero_rows(r):
            for jv in range(d_vecs):
                rows[0][r, pl.ds(jv * LANES, LANES)] = zvec

        row_base = pl.multiple_of(sid * stripe_lo, ZC)
        n_zc = jnp.where(sid == NS - 1, (n - (NS - 1) * stripe_lo) // ZC,
                         stripe_lo // ZC)

        @pl.loop(0, n_zc)
        def _zero_acc(t):
            off = pl.multiple_of(row_base + t * ZC, ZC)
            pltpu.sync_copy(rows[0].at[pl.ds(0, ZC)], acc.at[pl.ds(off, ZC)])

        plsc.subcore_barrier()

        def scale_rows(buf, adjref, nrows):
            @pl.loop(0, nrows // LANES)
            def _scale(q):
                a16 = adjref[pl.ds(q * LANES, LANES)]
                for t in range(LANES):
                    av = jnp.full((LANES,), a16[t], jnp.float32)
                    row = q * LANES + t
                    for jv in range(d_vecs):
                        sl = pl.ds(jv * LANES, LANES)
                        buf[row, sl] = buf[row, sl] * av

        def scale_tail8(buf, adjref):
            a16 = adjref[pl.ds(K - LANES, LANES)]
            for t in range(LANES // 2, LANES):
                av = jnp.full((LANES,), a16[t], jnp.float32)
                row = K - LANES + t
                for jv in range(d_vecs):
                    sl = pl.ds(jv * LANES, LANES)
                    buf[row, sl] = buf[row, sl] * av

        def stage(g, m3, m6):
            off = pl.multiple_of(ebase + g * K, 8)
            pltpu.async_copy(src_hbm.at[pl.ds(off, K)], srcb[m3], stsem[m3])
            pltpu.async_copy(adj_hbm.at[pl.ds(off, K)], adjb[m3], stsem[m3])
            pltpu.async_copy(dst_hbm.at[pl.ds(off, K)], dstb[m6].at[0], dsem[m6])

        def gathers(s3):
            pltpu.make_async_copy(src_hbm.at[pl.ds(0, K)], srcb[s3],
                                  stsem[s3]).wait()
            pltpu.make_async_copy(adj_hbm.at[pl.ds(0, K)], adjb[s3],
                                  stsem[s3]).wait()
            pltpu.async_copy(x_hbm.at[srcb[s3].at[pl.ds(0, K)]],
                             rows[s3], gsem[s3])

        def process(s3, m6):
            pltpu.make_async_copy(x_hbm.at[srcb[s3].at[pl.ds(0, K)]],
                                  rows[s3], gsem[s3]).wait()
            scale_rows(rows[s3], adjb[s3], K // LANES * LANES)
            scale_tail8(rows[s3], adjb[s3])
            pltpu.make_async_copy(dst_hbm.at[pl.ds(0, K)],
                                  dstb[m6].at[0], dsem[m6]).wait()
            pltpu.async_copy(rows[s3], acc.at[dstb[m6].at[0]],
                             scsem[s3], add=True)

        def drain(s3, m6):
            pltpu.make_async_copy(rows[s3], acc.at[dstb[m6].at[0]],
                                  scsem[s3]).wait()

        stage(0, 0, 0)
        stage(1, 1, 1)

        NIT = NPH // 6

        @pl.loop(0, NIT)
        def _pipe(gg):
            for j in range(6):
                g = 6 * gg + j
                s3 = j % 3
                if j >= 3:
                    drain(s3, (j + 3) % 6)
                else:
                    @pl.when(gg > 0)
                    def _d():
                        drain(s3, (j + 3) % 6)

                gathers(s3)

                if j == 0:
                    @pl.when(gg > 0)
                    def _p():
                        process(2, 5)
                else:
                    process((j - 1) % 3, (j - 1) % 6)

                if j >= 4:
                    @pl.when(gg < NIT - 1)
                    def _s():
                        stage(g + 2, (j + 2) % 3, (j + 2) % 6)
                else:
                    stage(g + 2, (j + 2) % 3, (j + 2) % 6)

        process(2, 5)
        drain(0, 3)
        drain(1, 4)
        drain(2, 5)

        if TE:
            to = pl.multiple_of(ebase + NPH * K, 8)
            pltpu.sync_copy(src_hbm.at[pl.ds(to, TE)],
                            srcb[0].at[pl.ds(0, TE)])
            pltpu.sync_copy(adj_hbm.at[pl.ds(to, TE)],
                            adjb[0].at[pl.ds(0, TE)])
            pltpu.sync_copy(dst_hbm.at[pl.ds(to, TE)], dstt.at[0])
            pltpu.async_copy(x_hbm.at[srcb[0].at[pl.ds(0, TE)]],
                             rows[0].at[pl.ds(0, TE)], gsem[0]).wait()
            scale_rows(rows[0], adjb[0], TE)
            pltpu.sync_copy(rows[0].at[pl.ds(0, TE)], acc.at[dstt.at[0]],
                            add=True)

        plsc.subcore_barrier()

        for t in range(stripe_lo // CP):
            off = pl.multiple_of(row_base + t * CP, ZC)
            pltpu.sync_copy(acc.at[pl.ds(off, CP)],
                            out_hbm.at[cid, pl.ds(off, CP)])

        @pl.when(sid == NS - 1)
        def _tail():
            off = (NS - 1) * stripe_lo + (stripe_lo // CP) * CP
            pltpu.sync_copy(acc.at[pl.ds(off, n - off)],
                            out_hbm.at[cid, pl.ds(off, n - off)])

    return k(x, src, dst, adj)


def _tc_combine_body(x_ref, agg_ref, w_ref, b_ref, out_ref):
    y = x_ref[...] + SMOOTH * (agg_ref[0] + agg_ref[1])
    out = jnp.dot(y, w_ref[...], preferred_element_type=jnp.float32)
    out_ref[...] = out / (1.0 + SMOOTH) + b_ref[...]


def _tc_combine(x, aggp, W, b, *, n, d_in, d_out):
    blk = 1000
    grid = n // blk
    return pl.pallas_call(
        _tc_combine_body,
        grid=(grid,),
        in_specs=[
            pl.BlockSpec((blk, d_in), lambda i: (i, 0)),
            pl.BlockSpec((NC, blk, d_in), lambda i: (0, i, 0)),
            pl.BlockSpec((d_in, d_out), lambda i: (0, 0)),
            pl.BlockSpec((1, d_out), lambda i: (0, 0)),
        ],
        out_specs=pl.BlockSpec((blk, d_out), lambda i: (i, 0)),
        out_shape=jax.ShapeDtypeStruct((n, d_out), jnp.float32),
    )(x, aggp, W, b)


def kernel(x, edge_index, adj_values, W, b):
    n, d_in = x.shape
    d_out = W.shape[1]
    e = edge_index.shape[1]
    src = edge_index[0].astype(jnp.int32)
    dst = edge_index[1].astype(jnp.int32)
    aggp = _sc_scatter(x, src, dst, adj_values, n=n, d=d_in, e=e)
    return _tc_combine(x, aggp, W, b.reshape(1, d_out), n=n, d_in=d_in, d_out=d_out)


# speedup vs baseline: 12.8501x; 1.0143x over previous
import functools

import jax
import jax.numpy as jnp
from jax import lax
from jax.experimental import pallas as pl
from jax.experimental.pallas import tpu as pltpu
from jax.experimental.pallas import tpu_sc as plsc

SMOOTH = 0.5

NC = 2
NS = 16
NW = NC * NS
LANES = 16


def _sc_scatter(x, src, dst, adj, *, n, d, e):
    per_w = e // NW
    K = 104
    NPH = per_w // K // 6 * 6
    TE = per_w - NPH * K
    ZC = 16
    stripe_lo = (n // NS) // ZC * ZC
    CP = 208
    d_vecs = d // LANES

    mesh = plsc.VectorSubcoreMesh(core_axis_name="c", subcore_axis_name="s")

    @functools.partial(
        pl.kernel,
        out_type=jax.ShapeDtypeStruct((NC, n, d), jnp.float32),
        mesh=mesh,
        scratch_types=[
            [pltpu.VMEM((K,), jnp.int32) for _ in range(3)],
            [pltpu.VMEM((K,), jnp.float32) for _ in range(3)],
            [pltpu.VMEM((1, K), jnp.int32) for _ in range(6)],
            [pltpu.VMEM((K, d), jnp.float32) for _ in range(3)],
            pltpu.VMEM((1, ZC), jnp.int32),
            pltpu.VMEM_SHARED((n, d), jnp.float32),
            [pltpu.SemaphoreType.DMA for _ in range(3)],
            [pltpu.SemaphoreType.DMA for _ in range(6)],
            [pltpu.SemaphoreType.DMA for _ in range(3)],
            [pltpu.SemaphoreType.DMA for _ in range(3)],
        ],
    )
    def k(x_hbm, src_hbm, dst_hbm, adj_hbm, out_hbm,
          srcb, adjb, dstb, rows, dstt, acc, stsem, dsem, gsem, scsem):
        cid = lax.axis_index("c")
        sid = lax.axis_index("s")
        wid = sid * NC + cid
        ebase = pl.multiple_of(wid * per_w, 8)

        zvec = jnp.zeros((LANES,), jnp.float32)

        @pl.loop(0, ZC)
        def _zero_rows(r):
            for jv in range(d_vecs):
                rows[0][r, pl.ds(jv * LANES, LANES)] = zvec

        row_base = pl.multiple_of(sid * stripe_lo, ZC)
        n_zc = jnp.where(sid == NS - 1, (n - (NS - 1) * stripe_lo) // ZC,
                         stripe_lo // ZC)

        @pl.loop(0, n_zc)
        def _zero_acc(t):
            off = pl.multiple_of(row_base + t * ZC, ZC)
            pltpu.async_copy(rows[0].at[pl.ds(0, ZC)], acc.at[pl.ds(off, ZC)],
                             gsem[0])

        @pl.loop(0, n_zc)
        def _zero_wait(t):
            off = pl.multiple_of(row_base + t * ZC, ZC)
            pltpu.make_async_copy(rows[0].at[pl.ds(0, ZC)],
                                  acc.at[pl.ds(off, ZC)], gsem[0]).wait()

        plsc.subcore_barrier()

        def scale_rows(buf, adjref, nrows):
            @pl.loop(0, nrows // LANES)
            def _scale(q):
                a16 = adjref[pl.ds(q * LANES, LANES)]
                for t in range(LANES):
                    av = jnp.full((LANES,), a16[t], jnp.float32)
                    row = q * LANES + t
                    for jv in range(d_vecs):
                        sl = pl.ds(jv * LANES, LANES)
                        buf[row, sl] = buf[row, sl] * av

        def scale_tail8(buf, adjref):
            a16 = adjref[pl.ds(K - LANES, LANES)]
            for t in range(LANES // 2, LANES):
                av = jnp.full((LANES,), a16[t], jnp.float32)
                row = K - LANES + t
                for jv in range(d_vecs):
                    sl = pl.ds(jv * LANES, LANES)
                    buf[row, sl] = buf[row, sl] * av

        def stage(g, m3, m6):
            off = pl.multiple_of(ebase + g * K, 8)
            pltpu.async_copy(src_hbm.at[pl.ds(off, K)], srcb[m3], stsem[m3])
            pltpu.async_copy(adj_hbm.at[pl.ds(off, K)], adjb[m3], stsem[m3])
            pltpu.async_copy(dst_hbm.at[pl.ds(off, K)], dstb[m6].at[0], dsem[m6])

        def gathers(s3):
            pltpu.make_async_copy(src_hbm.at[pl.ds(0, K)], srcb[s3],
                                  stsem[s3]).wait()
            pltpu.make_async_copy(adj_hbm.at[pl.ds(0, K)], adjb[s3],
                                  stsem[s3]).wait()
            pltpu.async_copy(x_hbm.at[srcb[s3].at[pl.ds(0, K)]],
                             rows[s3], gsem[s3])

        def process(s3, m6):
            pltpu.make_async_copy(x_hbm.at[srcb[s3].at[pl.ds(0, K)]],
                                  rows[s3], gsem[s3]).wait()
            scale_rows(rows[s3], adjb[s3], K // LANES * LANES)
            scale_tail8(rows[s3], adjb[s3])
            pltpu.make_async_copy(dst_hbm.at[pl.ds(0, K)],
                                  dstb[m6].at[0], dsem[m6]).wait()
            pltpu.async_copy(rows[s3], acc.at[dstb[m6].at[0]],
                             scsem[s3], add=True)

        def drain(s3, m6):
            pltpu.make_async_copy(rows[s3], acc.at[dstb[m6].at[0]],
                                  scsem[s3]).wait()

        stage(0, 0, 0)
        stage(1, 1, 1)

        NIT = NPH // 6

        @pl.loop(0, NIT)
        def _pipe(gg):
            for j in range(6):
                g = 6 * gg + j
                s3 = j % 3
                if j >= 3:
                    drain(s3, (j + 3) % 6)
                else:
                    @pl.when(gg > 0)
                    def _d():
                        drain(s3, (j + 3) % 6)

                gathers(s3)

                if j == 0:
                    @pl.when(gg > 0)
                    def _p():
                        process(2, 5)
                else:
                    process((j - 1) % 3, (j - 1) % 6)

                if j >= 4:
                    @pl.when(gg < NIT - 1)
                    def _s():
                        stage(g + 2, (j + 2) % 3, (j + 2) % 6)
                else:
                    stage(g + 2, (j + 2) % 3, (j + 2) % 6)

        process(2, 5)
        drain(0, 3)
        drain(1, 4)
        drain(2, 5)

        if TE:
            to = pl.multiple_of(ebase + NPH * K, 8)
            pltpu.sync_copy(src_hbm.at[pl.ds(to, TE)],
                            srcb[0].at[pl.ds(0, TE)])
            pltpu.sync_copy(adj_hbm.at[pl.ds(to, TE)],
                            adjb[0].at[pl.ds(0, TE)])
            pltpu.sync_copy(dst_hbm.at[pl.ds(to, TE)], dstt.at[0])
            pltpu.async_copy(x_hbm.at[srcb[0].at[pl.ds(0, TE)]],
                             rows[0].at[pl.ds(0, TE)], gsem[0]).wait()
            scale_rows(rows[0], adjb[0], TE)
            pltpu.sync_copy(rows[0].at[pl.ds(0, TE)], acc.at[dstt.at[0]],
                            add=True)

        plsc.subcore_barrier()

        for t in range(stripe_lo // CP):
            off = pl.multiple_of(row_base + t * CP, ZC)
            pltpu.async_copy(acc.at[pl.ds(off, CP)],
                             out_hbm.at[cid, pl.ds(off, CP)], scsem[t])

        @pl.when(sid == NS - 1)
        def _tail():
            off = (NS - 1) * stripe_lo + (stripe_lo // CP) * CP
            pltpu.sync_copy(acc.at[pl.ds(off, n - off)],
                            out_hbm.at[cid, pl.ds(off, n - off)])

        for t in range(stripe_lo // CP):
            off = pl.multiple_of(row_base + t * CP, ZC)
            pltpu.make_async_copy(acc.at[pl.ds(off, CP)],
                                  out_hbm.at[cid, pl.ds(off, CP)],
                                  scsem[t]).wait()

    return k(x, src, dst, adj)


def _tc_combine_body(x_ref, agg_ref, w_ref, b_ref, out_ref):
    y = x_ref[...] + SMOOTH * (agg_ref[0] + agg_ref[1])
    out = jnp.dot(y, w_ref[...], preferred_element_type=jnp.float32)
    out_ref[...] = out / (1.0 + SMOOTH) + b_ref[...]


def _tc_combine(x, aggp, W, b, *, n, d_in, d_out):
    blk = 1000
    grid = n // blk
    return pl.pallas_call(
        _tc_combine_body,
        grid=(grid,),
        in_specs=[
            pl.BlockSpec((blk, d_in), lambda i: (i, 0)),
            pl.BlockSpec((NC, blk, d_in), lambda i: (0, i, 0)),
            pl.BlockSpec((d_in, d_out), lambda i: (0, 0)),
            pl.BlockSpec((1, d_out), lambda i: (0, 0)),
        ],
        out_specs=pl.BlockSpec((blk, d_out), lambda i: (i, 0)),
        out_shape=jax.ShapeDtypeStruct((n, d_out), jnp.float32),
    )(x, aggp, W, b)


def kernel(x, edge_index, adj_values, W, b):
    n, d_in = x.shape
    d_out = W.shape[1]
    e = edge_index.shape[1]
    src = edge_index[0].astype(jnp.int32)
    dst = edge_index[1].astype(jnp.int32)
    aggp = _sc_scatter(x, src, dst, adj_values, n=n, d=d_in, e=e)
    return _tc_combine(x, aggp, W, b.reshape(1, d_out), n=n, d_in=d_in, d_out=d_out)
